# baseline (speedup 1.0000x reference)
import functools

import jax
import jax.numpy as jnp
from jax import lax
from jax.experimental import pallas as pl
from jax.experimental.pallas import tpu as pltpu
from jax.experimental.pallas import tpu_sc as plsc

B = 1024
L = 200
D = 128
N = B * L
NW = 32
PER_W = N // NW
CHUNK = 80
NCHUNK = PER_W // CHUNK
NG = CHUNK // 16
NLANE = 16
NBUF = 3


def _emb_body(x_hbm, seg_hbm, tok_hbm, segtab_hbm, postab_hbm, out_hbm,
              tidx, sidx, tok_a, tok_b, tok_c, ps3, seg_buf,
              gsem_a, gsem_b, gsem_c, wsem_a, wsem_b, wsem_c):
    wid = lax.axis_index("s") * 2 + lax.axis_index("c")
    base0 = wid * PER_W

    pltpu.sync_copy(x_hbm.at[pl.ds(base0, PER_W)], tidx)
    pltpu.sync_copy(seg_hbm.at[pl.ds(base0, PER_W)], sidx)
    pltpu.sync_copy(segtab_hbm, seg_buf)
    for s in range(3):
        pltpu.sync_copy(postab_hbm.at[pl.ds(0, L)],
                        ps3.at[pl.ds(s * L, L)])

    @plsc.parallel_loop(0, L)
    def ps3_body(i):
        for s in range(3):
            row = s * L + i
            for j in range(D // NLANE):
                sl = pl.ds(j * NLANE, NLANE)
                ps3[row, sl] = ps3[row, sl] + seg_buf[s, sl]

    lanes = jax.lax.iota(jnp.int32, NLANE)
    bufs = [tok_a, tok_b, tok_c]
    gsems = [gsem_a, gsem_b, gsem_c]
    wsems = [wsem_a, wsem_b, wsem_c]

    def issue_gather(k, b):
        pltpu.async_copy(tok_hbm.at[tidx.at[pl.ds(k * CHUNK, CHUNK)]],
                         bufs[b], gsems[b])

    def drain(sem):
        pltpu.make_async_copy(out_hbm.at[pl.ds(0, CHUNK)], bufs[0],
                              sem).wait()

    def process(k, b):
        lbase = k * CHUNK
        base = base0 + lbase
        buf = bufs[b]
        nb = (b + 2) % NBUF

        @pl.when(k + 2 < NCHUNK)
        def _():
            @pl.when(k >= 1)
            def _():
                drain(wsems[nb])
            issue_gather(k + 2, nb)

        rows_g = []
        psrow_g = []
        for g in range(NG):
            rows = lanes + g * NLANE
            lvec = lax.rem(rows + base, jnp.int32(L))
            sv = sidx[pl.ds(lbase + g * NLANE, NLANE)]
            rows_g.append(rows)
            psrow_g.append(sv * jnp.int32(L) + lvec)

        drain(gsems[b])

        @plsc.parallel_loop(0, D, unroll=8)
        def col_body(c):
            colv = lax.bitwise_and(lanes + c, jnp.int32(D - 1))
            for g in range(NG):
                p = plsc.load_gather(ps3, [psrow_g[g], colv])
                plsc.addupdate_scatter(buf, [rows_g[g], colv], p)

        pltpu.async_copy(buf, out_hbm.at[pl.ds(base, CHUNK)], wsems[b])

    issue_gather(0, 0)
    issue_gather(1, 1)

    def chunk_triple(ci3, _):
        for b in range(NBUF):
            process(ci3 * NBUF + b, b)
        return 0

    lax.fori_loop(0, NCHUNK // NBUF, chunk_triple, 0)
    for k in range((NCHUNK // NBUF) * NBUF, NCHUNK):
        process(k, k % NBUF)
    for b in range(NBUF):
        drain(wsems[(NCHUNK - 1 - b) % NBUF])


def kernel(x, x_segment, token_table, segment_table, position_table):
    mesh = plsc.VectorSubcoreMesh(core_axis_name="c", subcore_axis_name="s")
    run = functools.partial(
        pl.kernel,
        mesh=mesh,
        compiler_params=pltpu.CompilerParams(needs_layout_passes=False),
        out_type=jax.ShapeDtypeStruct((N, D), jnp.float32),
        scratch_types=[
            pltpu.VMEM((PER_W,), jnp.int32),
            pltpu.VMEM((PER_W,), jnp.int32),
            pltpu.VMEM((CHUNK, D), jnp.float32),
            pltpu.VMEM((CHUNK, D), jnp.float32),
            pltpu.VMEM((CHUNK, D), jnp.float32),
            pltpu.VMEM((3 * L, D), jnp.float32),
            pltpu.VMEM((3, D), jnp.float32),
            pltpu.SemaphoreType.DMA,
            pltpu.SemaphoreType.DMA,
            pltpu.SemaphoreType.DMA,
            pltpu.SemaphoreType.DMA,
            pltpu.SemaphoreType.DMA,
            pltpu.SemaphoreType.DMA,
        ],
    )(_emb_body)
    out = run(
        x.reshape(N),
        x_segment.reshape(N),
        token_table,
        segment_table,
        position_table,
    )
    return out.reshape(B, L, D)


# speedup vs baseline: 1.0153x; 1.0153x over previous
import functools

import jax
import jax.numpy as jnp
from jax import lax
from jax.experimental import pallas as pl
from jax.experimental.pallas import tpu as pltpu
from jax.experimental.pallas import tpu_sc as plsc

B = 1024
L = 200
D = 128
N = B * L
NW = 32
PER_W = N // NW
CHUNK = 128
NCHUNK = PER_W // CHUNK
NG = CHUNK // 16
NLANE = 16


def _emb_body(x_hbm, seg_hbm, tok_hbm, segtab_hbm, postab_hbm, out_hbm,
              tidx, sidx, tok_a, tok_b, ps3, seg_buf,
              gsem_a, gsem_b, wsem_a, wsem_b):
    wid = lax.axis_index("s") * 2 + lax.axis_index("c")
    base0 = wid * PER_W

    pltpu.sync_copy(x_hbm.at[pl.ds(base0, PER_W)], tidx)
    pltpu.sync_copy(seg_hbm.at[pl.ds(base0, PER_W)], sidx)
    pltpu.sync_copy(segtab_hbm, seg_buf)
    for s in range(3):
        pltpu.sync_copy(postab_hbm.at[pl.ds(0, L)],
                        ps3.at[pl.ds(s * L, L)])

    @plsc.parallel_loop(0, L)
    def ps3_body(i):
        for s in range(3):
            row = s * L + i
            for j in range(D // NLANE):
                sl = pl.ds(j * NLANE, NLANE)
                ps3[row, sl] = ps3[row, sl] + seg_buf[s, sl]

    lanes = jax.lax.iota(jnp.int32, NLANE)
    bufs = [tok_a, tok_b]
    gsems = [gsem_a, gsem_b]
    wsems = [wsem_a, wsem_b]

    def issue_gather(k, b):
        pltpu.async_copy(tok_hbm.at[tidx.at[pl.ds(k * CHUNK, CHUNK)]],
                         bufs[b], gsems[b])

    def drain(sem):
        pltpu.make_async_copy(out_hbm.at[pl.ds(0, CHUNK)], bufs[0],
                              sem).wait()

    issue_gather(0, 0)

    def chunk_pair(ci2, _):
        for b in range(2):
            k = ci2 * 2 + b
            lbase = k * CHUNK
            base = base0 + lbase
            buf = bufs[b]

            @pl.when(k + 1 < NCHUNK)
            def _():
                @pl.when(k >= 1)
                def _():
                    drain(wsems[1 - b])
                issue_gather(k + 1, 1 - b)

            rows_g = []
            psrow_g = []
            for g in range(NG):
                rows = lanes + g * NLANE
                lvec = lax.rem(rows + base, jnp.int32(L))
                sv = sidx[pl.ds(lbase + g * NLANE, NLANE)]
                rows_g.append(rows)
                psrow_g.append(sv * jnp.int32(L) + lvec)

            drain(gsems[b])

            @plsc.parallel_loop(0, D, unroll=8)
            def col_body(c):
                colv = lax.bitwise_and(lanes + c, jnp.int32(D - 1))
                for g in range(NG):
                    p = plsc.load_gather(ps3, [psrow_g[g], colv])
                    plsc.addupdate_scatter(buf, [rows_g[g], colv], p)

            pltpu.async_copy(buf, out_hbm.at[pl.ds(base, CHUNK)], wsems[b])
        return 0

    lax.fori_loop(0, NCHUNK // 2, chunk_pair, 0)
    drain(wsems[0])
    drain(wsems[1])


def kernel(x, x_segment, token_table, segment_table, position_table):
    mesh = plsc.VectorSubcoreMesh(core_axis_name="c", subcore_axis_name="s")
    run = functools.partial(
        pl.kernel,
        mesh=mesh,
        compiler_params=pltpu.CompilerParams(needs_layout_passes=False),
        out_type=jax.ShapeDtypeStruct((N, D), jnp.float32),
        scratch_types=[
            pltpu.VMEM((PER_W,), jnp.int32),
            pltpu.VMEM((PER_W,), jnp.int32),
            pltpu.VMEM((CHUNK, D), jnp.float32),
            pltpu.VMEM((CHUNK, D), jnp.float32),
            pltpu.VMEM((3 * L, D), jnp.float32),
            pltpu.VMEM((3, D), jnp.float32),
            pltpu.SemaphoreType.DMA,
            pltpu.SemaphoreType.DMA,
            pltpu.SemaphoreType.DMA,
            pltpu.SemaphoreType.DMA,
        ],
    )(_emb_body)
    out = run(
        x.reshape(N),
        x_segment.reshape(N),
        token_table,
        segment_table,
        position_table,
    )
    return out.reshape(B, L, D)
